# baseline (speedup 1.0000x reference)
import dataclasses
import functools

import jax
import jax.numpy as jnp
from jax import lax
from jax.experimental import pallas as pl
from jax.experimental.pallas import tpu as pltpu
from jax.experimental.pallas import tpu_sc as plsc

_N = 8 * 512 * 512
_K = _N // 4
_R = 4096
_C = 512
_CHUNK = 256
_NCH = _R // _CHUNK

_NTEC = 32
_BLK = 2048
_GRID = _N // _BLK
_H1 = 32768
_H2 = 65536

_sc_mesh = plsc.VectorSubcoreMesh(core_axis_name="c", subcore_axis_name="s")


def _sc_params():
    cp = pltpu.CompilerParams()
    if "needs_layout_passes" in pltpu.CompilerParams.__dataclass_fields__:
        cp = dataclasses.replace(cp, needs_layout_passes=False)
    return cp


def _loss_bits_body(x_ref, y_ref, o_ref):
    def chunk(i, carry):
        sl = pl.ds(i * _CHUNK, _CHUNK)
        x = x_ref[sl, :]
        y = y_ref[sl, :]
        loss = jnp.maximum(x, 0.0) - x * y + jnp.log1p(jnp.exp(-jnp.abs(x)))
        o_ref[sl, :] = lax.bitcast_convert_type(loss, jnp.int32)
        return carry

    lax.fori_loop(0, _NCH, chunk, 0)


def _sc_hist1(bits_flat):
    @functools.partial(
        pl.kernel,
        out_type=jax.ShapeDtypeStruct((_NTEC, _H1), jnp.int32),
        mesh=_sc_mesh,
        compiler_params=_sc_params(),
        scratch_types=[pltpu.VMEM((_H1,), jnp.int32)],
    )
    def k(bits_hbm, out_hbm, hist_v):
        wid = lax.axis_index("s") * 2 + lax.axis_index("c")

        @pl.loop(0, _H1, step=16)
        def _(i):
            hist_v[pl.ds(i, 16)] = jnp.zeros((16,), jnp.int32)

        ones = jnp.ones((16,), jnp.int32)

        def body(in_v):
            @pl.loop(0, _BLK, step=16)
            def _(c):
                v = in_v[pl.ds(c, 16)]
                plsc.addupdate_scatter(hist_v, [v >> 16], ones)

        pltpu.emit_pipeline(
            body,
            grid=(_GRID,),
            in_specs=[pl.BlockSpec((_BLK,), lambda i: (i,))],
            out_specs=[],
            core_axis_name=("c", "s"),
            dimension_semantics=(pltpu.PARALLEL,),
        )(bits_hbm)
        pltpu.sync_copy(hist_v, out_hbm.at[wid])

    return k(bits_flat)


def _find_bucket1_body(h_ref, o_ref):
    def merge(t, acc):
        return acc + h_ref[pl.ds(t * 256, 256), :]

    cnt = lax.fori_loop(0, _NTEC, merge, jnp.zeros((256, 128), jnp.int32))
    bidx = (
        lax.broadcasted_iota(jnp.int32, (256, 128), 0) * 128
        + lax.broadcasted_iota(jnp.int32, (256, 128), 1)
    )

    def count_gt(b):
        return jnp.sum(jnp.where(bidx > b, cnt, 0))

    def bisect(_, carry):
        lo, hi, c_hi = carry
        mid = lo + (hi - lo) // 2
        c = count_gt(mid)
        pred = c >= _K
        return (
            jnp.where(pred, mid, lo),
            jnp.where(pred, hi, mid),
            jnp.where(pred, c_hi, c),
        )

    lo, hi, c_hi = lax.fori_loop(
        0, 16, bisect, (jnp.int32(-1), jnp.int32(_H1), jnp.int32(0))
    )
    o_ref[0, 0] = hi
    o_ref[0, 1] = c_hi


def _sc_hist2(bits_flat, hstar16):
    @functools.partial(
        pl.kernel,
        out_type=jax.ShapeDtypeStruct((_NTEC, _H2), jnp.int32),
        mesh=_sc_mesh,
        compiler_params=_sc_params(),
        scratch_types=[
            pltpu.VMEM((_H2,), jnp.int32),
            pltpu.VMEM((16,), jnp.int32),
        ],
    )
    def k(bits_hbm, h_hbm, out_hbm, hist_v, h_v):
        wid = lax.axis_index("s") * 2 + lax.axis_index("c")
        pltpu.sync_copy(h_hbm, h_v)

        @pl.loop(0, _H2, step=16)
        def _(i):
            hist_v[pl.ds(i, 16)] = jnp.zeros((16,), jnp.int32)

        ones = jnp.ones((16,), jnp.int32)
        hstar = h_v[...]

        def body(in_v):
            @pl.loop(0, _BLK, step=16)
            def _(c):
                v = in_v[pl.ds(c, 16)]
                m = (v >> 16) == hstar
                plsc.addupdate_scatter(hist_v, [v & 0xFFFF], ones, mask=m)

        pltpu.emit_pipeline(
            body,
            grid=(_GRID,),
            in_specs=[pl.BlockSpec((_BLK,), lambda i: (i,))],
            out_specs=[],
            core_axis_name=("c", "s"),
            dimension_semantics=(pltpu.PARALLEL,),
        )(bits_hbm)
        pltpu.sync_copy(hist_v, out_hbm.at[wid])

    return k(bits_flat, hstar16)


def _finalize_body(h2_ref, bits_ref, hc_ref, o_ref):
    hstar = hc_ref[0, 0]
    c1 = hc_ref[0, 1]

    def merge(t, acc):
        return acc + h2_ref[pl.ds(t * 512, 512), :]

    cnt = lax.fori_loop(0, _NTEC, merge, jnp.zeros((512, 128), jnp.int32))
    bidx = (
        lax.broadcasted_iota(jnp.int32, (512, 128), 0) * 128
        + lax.broadcasted_iota(jnp.int32, (512, 128), 1)
    )

    def count_gt(b):
        return jnp.sum(jnp.where(bidx > b, cnt, 0))

    k_rem = _K - c1

    def bisect(_, carry):
        lo, hi, c_hi = carry
        mid = lo + (hi - lo) // 2
        c = count_gt(mid)
        pred = c >= k_rem
        return (
            jnp.where(pred, mid, lo),
            jnp.where(pred, hi, mid),
            jnp.where(pred, c_hi, c),
        )

    lo, hi, c2 = lax.fori_loop(
        0, 17, bisect, (jnp.int32(-1), jnp.int32(_H2), jnp.int32(0))
    )

    t_bits = (hstar << 16) | hi
    t_val = lax.bitcast_convert_type(t_bits, jnp.float32)

    def sum_gt(i, acc):
        b = bits_ref[pl.ds(i * _CHUNK, _CHUNK), :]
        v = lax.bitcast_convert_type(b, jnp.float32)
        return acc + jnp.sum(jnp.where(b > t_bits, v, 0.0))

    s = lax.fori_loop(0, _NCH, sum_gt, jnp.float32(0.0))

    mean = (s + (_K - c1 - c2).astype(jnp.float32) * t_val) / jnp.float32(_K)
    o_ref[0, 0] = mean


def kernel(logits, labels):
    x = logits.reshape(_R, _C)
    y = labels.astype(jnp.float32).reshape(_R, _C)

    bits = pl.pallas_call(
        _loss_bits_body,
        out_shape=jax.ShapeDtypeStruct((_R, _C), jnp.int32),
    )(x, y)

    hist1 = _sc_hist1(bits.reshape(_N))

    hc = pl.pallas_call(
        _find_bucket1_body,
        out_shape=jax.ShapeDtypeStruct((1, 2), jnp.int32),
        out_specs=pl.BlockSpec(memory_space=pltpu.SMEM),
    )(hist1.reshape(_NTEC * 256, 128))

    hstar16 = jnp.broadcast_to(hc[0, 0], (16,))
    hist2 = _sc_hist2(bits.reshape(_N), hstar16)

    out = pl.pallas_call(
        _finalize_body,
        out_shape=jax.ShapeDtypeStruct((1, 1), jnp.float32),
        in_specs=[
            pl.BlockSpec(memory_space=pltpu.VMEM),
            pl.BlockSpec(memory_space=pltpu.VMEM),
            pl.BlockSpec(memory_space=pltpu.SMEM),
        ],
        out_specs=pl.BlockSpec(memory_space=pltpu.SMEM),
    )(hist2.reshape(_NTEC * 512, 128), bits, hc)

    return out[0, 0]


# speedup vs baseline: 1.7435x; 1.7435x over previous
import dataclasses
import functools

import jax
import jax.numpy as jnp
from jax import lax
from jax.experimental import pallas as pl
from jax.experimental.pallas import tpu as pltpu
from jax.experimental.pallas import tpu_sc as plsc

_N = 8 * 512 * 512
_K = _N // 4
_R = 4096
_C = 512
_CHUNK = 256
_NCH = _R // _CHUNK

_NTEC = 32
_BLK = 16384
_GRID = _N // _BLK
_H1 = 32768
_H2 = 65536

_sc_mesh = plsc.VectorSubcoreMesh(core_axis_name="c", subcore_axis_name="s")


def _sc_params():
    cp = pltpu.CompilerParams()
    if "needs_layout_passes" in pltpu.CompilerParams.__dataclass_fields__:
        cp = dataclasses.replace(cp, needs_layout_passes=False)
    return cp


def _loss_bits_body(x_ref, y_ref, o_ref):
    def chunk(i, carry):
        sl = pl.ds(i * _CHUNK, _CHUNK)
        x = x_ref[sl, :]
        y = y_ref[sl, :]
        loss = jnp.maximum(x, 0.0) - x * y + jnp.log1p(jnp.exp(-jnp.abs(x)))
        o_ref[sl, :] = lax.bitcast_convert_type(loss, jnp.int32)
        return carry

    lax.fori_loop(0, _NCH, chunk, 0)


def _sc_hist1(bits_flat):
    @functools.partial(
        pl.kernel,
        out_type=jax.ShapeDtypeStruct((_NTEC, _H1), jnp.int32),
        mesh=_sc_mesh,
        compiler_params=_sc_params(),
        scratch_types=[pltpu.VMEM((_H1,), jnp.int32)],
    )
    def k(bits_hbm, out_hbm, hist_v):
        wid = lax.axis_index("s") * 2 + lax.axis_index("c")

        @plsc.parallel_loop(0, _H1, step=16, unroll=8)
        def _(i):
            hist_v[pl.ds(i, 16)] = jnp.zeros((16,), jnp.int32)

        ones = jnp.ones((16,), jnp.int32)

        def body(in_v):
            @plsc.parallel_loop(0, _BLK, step=16, unroll=8)
            def _(c):
                v = in_v[pl.ds(c, 16)]
                plsc.addupdate_scatter(hist_v, [v >> 16], ones)

        pltpu.emit_pipeline(
            body,
            grid=(_GRID,),
            in_specs=[pl.BlockSpec((_BLK,), lambda i: (i,))],
            out_specs=[],
            core_axis_name=("c", "s"),
            dimension_semantics=(pltpu.PARALLEL,),
        )(bits_hbm)
        pltpu.sync_copy(hist_v, out_hbm.at[wid])

    return k(bits_flat)


def _find_bucket1_body(h_ref, o_ref):
    def merge(t, acc):
        return acc + h_ref[pl.ds(t * 256, 256), :]

    cnt = lax.fori_loop(0, _NTEC, merge, jnp.zeros((256, 128), jnp.int32))
    bidx = (
        lax.broadcasted_iota(jnp.int32, (256, 128), 0) * 128
        + lax.broadcasted_iota(jnp.int32, (256, 128), 1)
    )

    def count_gt(b):
        return jnp.sum(jnp.where(bidx > b, cnt, 0))

    def bisect(_, carry):
        lo, hi, c_hi = carry
        mid = lo + (hi - lo) // 2
        c = count_gt(mid)
        pred = c >= _K
        return (
            jnp.where(pred, mid, lo),
            jnp.where(pred, hi, mid),
            jnp.where(pred, c_hi, c),
        )

    lo, hi, c_hi = lax.fori_loop(
        0, 16, bisect, (jnp.int32(-1), jnp.int32(_H1), jnp.int32(0))
    )
    o_ref[0, 0] = hi
    o_ref[0, 1] = c_hi


def _sc_hist2(bits_flat, hstar16):
    @functools.partial(
        pl.kernel,
        out_type=jax.ShapeDtypeStruct((_NTEC, _H2), jnp.int32),
        mesh=_sc_mesh,
        compiler_params=_sc_params(),
        scratch_types=[
            pltpu.VMEM((_H2,), jnp.int32),
            pltpu.VMEM((16,), jnp.int32),
        ],
    )
    def k(bits_hbm, h_hbm, out_hbm, hist_v, h_v):
        wid = lax.axis_index("s") * 2 + lax.axis_index("c")
        pltpu.sync_copy(h_hbm, h_v)

        @plsc.parallel_loop(0, _H2, step=16, unroll=8)
        def _(i):
            hist_v[pl.ds(i, 16)] = jnp.zeros((16,), jnp.int32)

        ones = jnp.ones((16,), jnp.int32)
        hstar = h_v[...]

        def body(in_v):
            @plsc.parallel_loop(0, _BLK, step=16, unroll=8)
            def _(c):
                v = in_v[pl.ds(c, 16)]
                m = (v >> 16) == hstar
                plsc.addupdate_scatter(hist_v, [v & 0xFFFF], ones, mask=m)

        pltpu.emit_pipeline(
            body,
            grid=(_GRID,),
            in_specs=[pl.BlockSpec((_BLK,), lambda i: (i,))],
            out_specs=[],
            core_axis_name=("c", "s"),
            dimension_semantics=(pltpu.PARALLEL,),
        )(bits_hbm)
        pltpu.sync_copy(hist_v, out_hbm.at[wid])

    return k(bits_flat, hstar16)


def _finalize_body(h2_ref, bits_ref, hc_ref, o_ref):
    hstar = hc_ref[0, 0]
    c1 = hc_ref[0, 1]

    def merge(t, acc):
        return acc + h2_ref[pl.ds(t * 512, 512), :]

    cnt = lax.fori_loop(0, _NTEC, merge, jnp.zeros((512, 128), jnp.int32))
    bidx = (
        lax.broadcasted_iota(jnp.int32, (512, 128), 0) * 128
        + lax.broadcasted_iota(jnp.int32, (512, 128), 1)
    )

    def count_gt(b):
        return jnp.sum(jnp.where(bidx > b, cnt, 0))

    k_rem = _K - c1

    def bisect(_, carry):
        lo, hi, c_hi = carry
        mid = lo + (hi - lo) // 2
        c = count_gt(mid)
        pred = c >= k_rem
        return (
            jnp.where(pred, mid, lo),
            jnp.where(pred, hi, mid),
            jnp.where(pred, c_hi, c),
        )

    lo, hi, c2 = lax.fori_loop(
        0, 17, bisect, (jnp.int32(-1), jnp.int32(_H2), jnp.int32(0))
    )

    t_bits = (hstar << 16) | hi
    t_val = lax.bitcast_convert_type(t_bits, jnp.float32)

    def sum_gt(i, acc):
        b = bits_ref[pl.ds(i * _CHUNK, _CHUNK), :]
        v = lax.bitcast_convert_type(b, jnp.float32)
        return acc + jnp.sum(jnp.where(b > t_bits, v, 0.0))

    s = lax.fori_loop(0, _NCH, sum_gt, jnp.float32(0.0))

    mean = (s + (_K - c1 - c2).astype(jnp.float32) * t_val) / jnp.float32(_K)
    o_ref[0, 0] = mean


def kernel(logits, labels):
    x = logits.reshape(_R, _C)
    y = labels.astype(jnp.float32).reshape(_R, _C)

    bits = pl.pallas_call(
        _loss_bits_body,
        out_shape=jax.ShapeDtypeStruct((_R, _C), jnp.int32),
    )(x, y)

    hist1 = _sc_hist1(bits.reshape(_N))

    hc = pl.pallas_call(
        _find_bucket1_body,
        out_shape=jax.ShapeDtypeStruct((1, 2), jnp.int32),
        out_specs=pl.BlockSpec(memory_space=pltpu.SMEM),
    )(hist1.reshape(_NTEC * 256, 128))

    hstar16 = jnp.broadcast_to(hc[0, 0], (16,))
    hist2 = _sc_hist2(bits.reshape(_N), hstar16)

    out = pl.pallas_call(
        _finalize_body,
        out_shape=jax.ShapeDtypeStruct((1, 1), jnp.float32),
        in_specs=[
            pl.BlockSpec(memory_space=pltpu.VMEM),
            pl.BlockSpec(memory_space=pltpu.VMEM),
            pl.BlockSpec(memory_space=pltpu.SMEM),
        ],
        out_specs=pl.BlockSpec(memory_space=pltpu.SMEM),
    )(hist2.reshape(_NTEC * 512, 128), bits, hc)

    return out[0, 0]


# speedup vs baseline: 1.9735x; 1.1319x over previous
import dataclasses
import functools

import jax
import jax.numpy as jnp
from jax import lax
from jax.experimental import pallas as pl
from jax.experimental.pallas import tpu as pltpu
from jax.experimental.pallas import tpu_sc as plsc

_N = 8 * 512 * 512
_K = _N // 4
_R = 16384
_C = 128
_CHUNK = 1024
_NCH = _R // _CHUNK

_NTEC = 32
_BLKR = 128
_GRID = _R // _BLKR
_H1 = 32768
_H2 = 65536

_sc_mesh = plsc.VectorSubcoreMesh(core_axis_name="c", subcore_axis_name="s")


def _sc_params():
    cp = pltpu.CompilerParams()
    if "needs_layout_passes" in pltpu.CompilerParams.__dataclass_fields__:
        cp = dataclasses.replace(cp, needs_layout_passes=False)
    return cp


def _loss_bits_body(x_ref, y_ref, o_ref):
    x = x_ref[...]
    y = y_ref[...].astype(jnp.float32)
    loss = jnp.maximum(x, 0.0) - x * y + jnp.log1p(jnp.exp(-jnp.abs(x)))
    o_ref[...] = lax.bitcast_convert_type(loss, jnp.int32)


def _sc_hist1(bits_flat):
    @functools.partial(
        pl.kernel,
        out_type=jax.ShapeDtypeStruct((_NTEC * _H1,), jnp.int32),
        mesh=_sc_mesh,
        compiler_params=_sc_params(),
        scratch_types=[pltpu.VMEM((_H1,), jnp.int32)],
    )
    def k(bits_hbm, out_hbm, hist_v):
        wid = lax.axis_index("s") * 2 + lax.axis_index("c")

        @plsc.parallel_loop(0, _H1, step=16, unroll=8)
        def _(i):
            hist_v[pl.ds(i, 16)] = jnp.zeros((16,), jnp.int32)

        ones = jnp.ones((16,), jnp.int32)

        def body(in_v):
            @plsc.parallel_loop(0, _BLKR, step=1, unroll=2)
            def _(r):
                for g in range(8):
                    v = in_v[r, pl.ds(g * 16, 16)]
                    plsc.addupdate_scatter(hist_v, [v >> 16], ones)

        pltpu.emit_pipeline(
            body,
            grid=(_GRID,),
            in_specs=[pl.BlockSpec((_BLKR, 128), lambda i: (i, 0))],
            out_specs=[],
            core_axis_name=("c", "s"),
            dimension_semantics=(pltpu.PARALLEL,),
        )(bits_hbm)
        pltpu.sync_copy(hist_v, out_hbm.at[pl.ds(wid * _H1, _H1)])

    return k(bits_flat)


def _find_bucket1_body(h_ref, o_ref, ovec_ref):
    def merge(t, acc):
        return acc + h_ref[pl.ds(t * 256, 256), :]

    cnt = lax.fori_loop(0, _NTEC, merge, jnp.zeros((256, 128), jnp.int32))
    bidx = (
        lax.broadcasted_iota(jnp.int32, (256, 128), 0) * 128
        + lax.broadcasted_iota(jnp.int32, (256, 128), 1)
    )

    def count_gt(b):
        return jnp.sum(jnp.where(bidx > b, cnt, 0))

    def bisect(_, carry):
        lo, hi, c_hi = carry
        mid = lo + (hi - lo) // 2
        c = count_gt(mid)
        pred = c >= _K
        return (
            jnp.where(pred, mid, lo),
            jnp.where(pred, hi, mid),
            jnp.where(pred, c_hi, c),
        )

    lo, hi, c_hi = lax.fori_loop(
        0, 16, bisect, (jnp.int32(-1), jnp.int32(_H1), jnp.int32(0))
    )
    o_ref[0, 0] = hi
    o_ref[0, 1] = c_hi
    for i in range(16):
        ovec_ref[i] = hi


def _sc_hist2(bits_flat, hstar16):
    @functools.partial(
        pl.kernel,
        out_type=jax.ShapeDtypeStruct((_NTEC * _H2,), jnp.int32),
        mesh=_sc_mesh,
        compiler_params=_sc_params(),
        scratch_types=[
            pltpu.VMEM((_H2,), jnp.int32),
            pltpu.VMEM((16,), jnp.int32),
        ],
    )
    def k(bits_hbm, h_hbm, out_hbm, hist_v, h_v):
        wid = lax.axis_index("s") * 2 + lax.axis_index("c")
        pltpu.sync_copy(h_hbm, h_v)

        @plsc.parallel_loop(0, _H2, step=16, unroll=8)
        def _(i):
            hist_v[pl.ds(i, 16)] = jnp.zeros((16,), jnp.int32)

        ones = jnp.ones((16,), jnp.int32)
        hstar = h_v[...]

        def body(in_v):
            @plsc.parallel_loop(0, _BLKR, step=1, unroll=2)
            def _(r):
                for g in range(8):
                    v = in_v[r, pl.ds(g * 16, 16)]
                    m = (v >> 16) == hstar
                    plsc.addupdate_scatter(hist_v, [v & 0xFFFF], ones, mask=m)

        pltpu.emit_pipeline(
            body,
            grid=(_GRID,),
            in_specs=[pl.BlockSpec((_BLKR, 128), lambda i: (i, 0))],
            out_specs=[],
            core_axis_name=("c", "s"),
            dimension_semantics=(pltpu.PARALLEL,),
        )(bits_hbm)
        pltpu.sync_copy(hist_v, out_hbm.at[pl.ds(wid * _H2, _H2)])

    return k(bits_flat, hstar16)


def _finalize_body(h2_ref, bits_ref, hc_ref, o_ref):
    hstar = hc_ref[0, 0]
    c1 = hc_ref[0, 1]

    def merge(t, acc):
        return acc + h2_ref[pl.ds(t * 512, 512), :]

    cnt = lax.fori_loop(0, _NTEC, merge, jnp.zeros((512, 128), jnp.int32))
    bidx = (
        lax.broadcasted_iota(jnp.int32, (512, 128), 0) * 128
        + lax.broadcasted_iota(jnp.int32, (512, 128), 1)
    )

    def count_gt(b):
        return jnp.sum(jnp.where(bidx > b, cnt, 0))

    k_rem = _K - c1

    def bisect(_, carry):
        lo, hi, c_hi = carry
        mid = lo + (hi - lo) // 2
        c = count_gt(mid)
        pred = c >= k_rem
        return (
            jnp.where(pred, mid, lo),
            jnp.where(pred, hi, mid),
            jnp.where(pred, c_hi, c),
        )

    lo, hi, c2 = lax.fori_loop(
        0, 17, bisect, (jnp.int32(-1), jnp.int32(_H2), jnp.int32(0))
    )

    t_bits = (hstar << 16) | hi
    t_val = lax.bitcast_convert_type(t_bits, jnp.float32)

    def sum_gt(i, acc):
        b = bits_ref[pl.ds(i * _CHUNK, _CHUNK), :]
        v = lax.bitcast_convert_type(b, jnp.float32)
        return acc + jnp.sum(jnp.where(b > t_bits, v, 0.0))

    s = lax.fori_loop(0, _NCH, sum_gt, jnp.float32(0.0))

    mean = (s + (_K - c1 - c2).astype(jnp.float32) * t_val) / jnp.float32(_K)
    o_ref[0, 0] = mean


def kernel(logits, labels):
    x = logits.reshape(_R, _C)
    y = labels.reshape(_R, _C)

    bits = pl.pallas_call(
        _loss_bits_body,
        grid=(_NCH,),
        in_specs=[
            pl.BlockSpec((_CHUNK, _C), lambda i: (i, 0)),
            pl.BlockSpec((_CHUNK, _C), lambda i: (i, 0)),
        ],
        out_specs=pl.BlockSpec((_CHUNK, _C), lambda i: (i, 0)),
        out_shape=jax.ShapeDtypeStruct((_R, _C), jnp.int32),
    )(x, y)

    hist1 = _sc_hist1(bits)

    hc, hvec = pl.pallas_call(
        _find_bucket1_body,
        out_shape=[
            jax.ShapeDtypeStruct((1, 2), jnp.int32),
            jax.ShapeDtypeStruct((16,), jnp.int32),
        ],
        out_specs=[
            pl.BlockSpec(memory_space=pltpu.SMEM),
            pl.BlockSpec(memory_space=pltpu.SMEM),
        ],
    )(hist1.reshape(_NTEC * 256, 128))

    hist2 = _sc_hist2(bits, hvec)

    out = pl.pallas_call(
        _finalize_body,
        out_shape=jax.ShapeDtypeStruct((1, 1), jnp.float32),
        in_specs=[
            pl.BlockSpec(memory_space=pltpu.VMEM),
            pl.BlockSpec(memory_space=pltpu.VMEM),
            pl.BlockSpec(memory_space=pltpu.SMEM),
        ],
        out_specs=pl.BlockSpec(memory_space=pltpu.SMEM),
    )(hist2.reshape(_NTEC * 512, 128), bits, hc)

    return out[0, 0]


# speedup vs baseline: 2.2214x; 1.1256x over previous
import dataclasses
import functools

import jax
import jax.numpy as jnp
from jax import lax
from jax.experimental import pallas as pl
from jax.experimental.pallas import tpu as pltpu
from jax.experimental.pallas import tpu_sc as plsc

_N = 8 * 512 * 512
_K = _N // 4
_R = 16384
_C = 128
_CHUNK = 1024
_NCH = _R // _CHUNK

_NTEC = 32
_BLKR = 128
_GRID = _R // _BLKR
_S1 = 18
_H1 = 8192
_S2 = 3
_H2 = 32768

_sc_mesh = plsc.VectorSubcoreMesh(core_axis_name="c", subcore_axis_name="s")


def _sc_params():
    cp = pltpu.CompilerParams()
    if "needs_layout_passes" in pltpu.CompilerParams.__dataclass_fields__:
        cp = dataclasses.replace(cp, needs_layout_passes=False)
    if "use_tc_tiling_on_sc" in pltpu.CompilerParams.__dataclass_fields__:
        cp = dataclasses.replace(cp, use_tc_tiling_on_sc=True)
    return cp


def _loss_bits_body(x_ref, y_ref, o_ref):
    def chunk(i, carry):
        sl = pl.ds(i * _CHUNK, _CHUNK)
        x = x_ref[sl, :]
        y = y_ref[sl, :].astype(jnp.float32)
        loss = jnp.maximum(x, 0.0) - x * y + jnp.log1p(jnp.exp(-jnp.abs(x)))
        o_ref[sl, :] = lax.bitcast_convert_type(loss, jnp.int32)
        return carry

    lax.fori_loop(0, _NCH, chunk, 0)


def _sc_hist1(bits2d):
    @functools.partial(
        pl.kernel,
        out_type=jax.ShapeDtypeStruct((_NTEC * _H1,), jnp.int32),
        mesh=_sc_mesh,
        compiler_params=_sc_params(),
        scratch_types=[pltpu.VMEM((_H1,), jnp.int32)],
    )
    def k(bits_hbm, out_hbm, hist_v):
        wid = lax.axis_index("s") * 2 + lax.axis_index("c")

        @plsc.parallel_loop(0, _H1, step=16, unroll=8)
        def _(i):
            hist_v[pl.ds(i, 16)] = jnp.zeros((16,), jnp.int32)

        ones = jnp.ones((16,), jnp.int32)

        def body(in_v):
            @plsc.parallel_loop(0, _BLKR, step=1, unroll=2)
            def _(r):
                for g in range(8):
                    v = in_v[r, pl.ds(g * 16, 16)]
                    plsc.addupdate_scatter(hist_v, [v >> _S1], ones)

        pltpu.emit_pipeline(
            body,
            grid=(_GRID,),
            in_specs=[pl.BlockSpec((_BLKR, 128), lambda i: (i, 0))],
            out_specs=[],
            core_axis_name=("c", "s"),
            dimension_semantics=(pltpu.PARALLEL,),
        )(bits_hbm)
        pltpu.sync_copy(hist_v, out_hbm.at[pl.ds(wid * _H1, _H1)])

    return k(bits2d)


def _find_bucket1_body(h_ref, o_ref, ovec_ref):
    def merge(t, acc):
        return acc + h_ref[pl.ds(t * (_H1 // 128), _H1 // 128), :]

    cnt = lax.fori_loop(
        0, _NTEC, merge, jnp.zeros((_H1 // 128, 128), jnp.int32)
    )
    bidx = (
        lax.broadcasted_iota(jnp.int32, (_H1 // 128, 128), 0) * 128
        + lax.broadcasted_iota(jnp.int32, (_H1 // 128, 128), 1)
    )

    def count_gt(b):
        return jnp.sum(jnp.where(bidx > b, cnt, 0))

    def bisect(_, carry):
        lo, hi, c_hi = carry
        mid = lo + (hi - lo) // 2
        c = count_gt(mid)
        pred = c >= _K
        return (
            jnp.where(pred, mid, lo),
            jnp.where(pred, hi, mid),
            jnp.where(pred, c_hi, c),
        )

    lo, hi, c_hi = lax.fori_loop(
        0, 14, bisect, (jnp.int32(-1), jnp.int32(_H1), jnp.int32(0))
    )
    o_ref[0, 0] = hi
    o_ref[0, 1] = c_hi
    for i in range(16):
        ovec_ref[i] = hi


def _sc_hist2(bits2d, hstar16):
    @functools.partial(
        pl.kernel,
        out_type=jax.ShapeDtypeStruct((_NTEC * _H2,), jnp.int32),
        mesh=_sc_mesh,
        compiler_params=_sc_params(),
        scratch_types=[
            pltpu.VMEM((_H2,), jnp.int32),
            pltpu.VMEM((16,), jnp.int32),
        ],
    )
    def k(bits_hbm, h_hbm, out_hbm, hist_v, h_v):
        wid = lax.axis_index("s") * 2 + lax.axis_index("c")
        pltpu.sync_copy(h_hbm, h_v)

        @plsc.parallel_loop(0, _H2, step=16, unroll=8)
        def _(i):
            hist_v[pl.ds(i, 16)] = jnp.zeros((16,), jnp.int32)

        ones = jnp.ones((16,), jnp.int32)
        hstar = h_v[...]

        def body(in_v):
            @plsc.parallel_loop(0, _BLKR, step=1, unroll=2)
            def _(r):
                for g in range(8):
                    v = in_v[r, pl.ds(g * 16, 16)]
                    m = (v >> _S1) == hstar
                    plsc.addupdate_scatter(
                        hist_v, [(v >> _S2) & (_H2 - 1)], ones, mask=m
                    )

        pltpu.emit_pipeline(
            body,
            grid=(_GRID,),
            in_specs=[pl.BlockSpec((_BLKR, 128), lambda i: (i, 0))],
            out_specs=[],
            core_axis_name=("c", "s"),
            dimension_semantics=(pltpu.PARALLEL,),
        )(bits_hbm)
        pltpu.sync_copy(hist_v, out_hbm.at[pl.ds(wid * _H2, _H2)])

    return k(bits2d, hstar16)


def _finalize_body(h2_ref, bits_ref, hc_ref, o_ref):
    hstar = hc_ref[0, 0]
    c1 = hc_ref[0, 1]

    def merge(t, acc):
        return acc + h2_ref[pl.ds(t * (_H2 // 128), _H2 // 128), :]

    cnt = lax.fori_loop(
        0, _NTEC, merge, jnp.zeros((_H2 // 128, 128), jnp.int32)
    )
    bidx = (
        lax.broadcasted_iota(jnp.int32, (_H2 // 128, 128), 0) * 128
        + lax.broadcasted_iota(jnp.int32, (_H2 // 128, 128), 1)
    )

    def count_gt(b):
        return jnp.sum(jnp.where(bidx > b, cnt, 0))

    k_rem = _K - c1

    def bisect(_, carry):
        lo, hi, c_hi = carry
        mid = lo + (hi - lo) // 2
        c = count_gt(mid)
        pred = c >= k_rem
        return (
            jnp.where(pred, mid, lo),
            jnp.where(pred, hi, mid),
            jnp.where(pred, c_hi, c),
        )

    lo, hi, c2 = lax.fori_loop(
        0, 16, bisect, (jnp.int32(-1), jnp.int32(_H2), jnp.int32(0))
    )

    t_lo = (hstar << _S1) | (hi << _S2)
    t_up = t_lo | ((1 << _S2) - 1)
    t_val = lax.bitcast_convert_type(t_lo, jnp.float32)

    def sum_gt(i, acc):
        b = bits_ref[pl.ds(i * _CHUNK, _CHUNK), :]
        v = lax.bitcast_convert_type(b, jnp.float32)
        return acc + jnp.sum(jnp.where(b > t_up, v, 0.0))

    s = lax.fori_loop(0, _NCH, sum_gt, jnp.float32(0.0))

    mean = (s + (_K - c1 - c2).astype(jnp.float32) * t_val) / jnp.float32(_K)
    o_ref[0, 0] = mean


def kernel(logits, labels):
    x = logits.reshape(_R, _C)
    y = labels.reshape(_R, _C)

    bits = pl.pallas_call(
        _loss_bits_body,
        out_shape=jax.ShapeDtypeStruct((_R, _C), jnp.int32),
    )(x, y)

    hist1 = _sc_hist1(bits)

    hc, hvec = pl.pallas_call(
        _find_bucket1_body,
        out_shape=[
            jax.ShapeDtypeStruct((1, 2), jnp.int32),
            jax.ShapeDtypeStruct((16,), jnp.int32),
        ],
        out_specs=[
            pl.BlockSpec(memory_space=pltpu.SMEM),
            pl.BlockSpec(memory_space=pltpu.SMEM),
        ],
    )(hist1.reshape(_NTEC * _H1 // 128, 128))

    hist2 = _sc_hist2(bits, hvec)

    out = pl.pallas_call(
        _finalize_body,
        out_shape=jax.ShapeDtypeStruct((1, 1), jnp.float32),
        in_specs=[
            pl.BlockSpec(memory_space=pltpu.VMEM),
            pl.BlockSpec(memory_space=pltpu.VMEM),
            pl.BlockSpec(memory_space=pltpu.SMEM),
        ],
        out_specs=pl.BlockSpec(memory_space=pltpu.SMEM),
    )(hist2.reshape(_NTEC * _H2 // 128, 128), bits, hc)

    return out[0, 0]
